# baseline (speedup 1.0000x reference)
import functools

import jax
import jax.numpy as jnp
from jax import lax
from jax.experimental import pallas as pl
from jax.experimental.pallas import tpu as pltpu
from jax.experimental.pallas import tpu_sc as plsc

_NC = 2
_NS = 16
_NW = _NC * _NS

_BT = 256
_BI = 1024
_BH = 1024


def _routing_metadata(top_k_index, top_k_weights, T, K, E, Pmax, NB):
    A = T * K
    e_flat = top_k_index.reshape(A).astype(jnp.int32)
    w_flat = top_k_weights.reshape(A).astype(jnp.float32)
    onehot = (e_flat[:, None] == jnp.arange(E, dtype=jnp.int32)[None, :]).astype(jnp.int32)
    rank = jnp.take_along_axis(jnp.cumsum(onehot, axis=0) - onehot,
                               e_flat[:, None], axis=1)[:, 0]
    counts = jnp.sum(onehot, axis=0)
    blocks_per_e = (counts + _BT - 1) // _BT
    block_end = jnp.cumsum(blocks_per_e)
    group_start = (block_end - blocks_per_e) * _BT
    pos = group_start[e_flat] + rank
    tok = jnp.arange(A, dtype=jnp.int32) // K
    src_tok = jnp.zeros((Pmax,), jnp.int32).at[pos].set(tok)
    w_row = jnp.zeros((Pmax,), jnp.float32).at[pos].set(w_flat)
    gid = jnp.searchsorted(block_end, jnp.arange(NB), side="right")
    gid = jnp.minimum(gid, E - 1).astype(jnp.int32)
    return pos.astype(jnp.int32), src_tok, w_row, gid


def _sc_gather(table, src_tok, Pmax, W, dtype):
    rows_per_w = Pmax // _NW
    CH = 32
    n_chunks = rows_per_w // CH
    mesh = plsc.VectorSubcoreMesh(core_axis_name="c", subcore_axis_name="s")

    @functools.partial(
        pl.kernel, mesh=mesh,
        out_type=jax.ShapeDtypeStruct((Pmax, W), dtype),
        scratch_types=[
            pltpu.VMEM((rows_per_w,), jnp.int32),
            pltpu.VMEM((2, CH, W), dtype),
            pltpu.SemaphoreType.DMA,
            pltpu.SemaphoreType.DMA,
        ],
    )
    def gather_k(hs_hbm, idx_hbm, out_hbm, idx_v, rows_v, sem0, sem1):
        wid = lax.axis_index("s") * _NC + lax.axis_index("c")
        base = wid * rows_per_w
        pltpu.sync_copy(idx_hbm.at[pl.ds(base, rows_per_w)], idx_v)
        sems = (sem0, sem1)
        cps = [None, None]

        def flush(j):
            b = j & 1
            cps[b].wait()
            pltpu.sync_copy(rows_v.at[b], out_hbm.at[pl.ds(base + j * CH, CH)])

        for i in range(n_chunks):
            b = i & 1
            cps[b] = pltpu.async_copy(
                hs_hbm.at[idx_v.at[pl.ds(i * CH, CH)]], rows_v.at[b], sems[b])
            if i >= 1:
                flush(i - 1)
        flush(n_chunks - 1)

    return gather_k(table, src_tok)


def _sc_combine(out_rows, pos, T, K, H):
    toks_per_w = T // _NW
    CT = 8
    n_chunks = toks_per_w // CT
    Wp = H // 2
    Qh = _BH // 2
    mesh = plsc.VectorSubcoreMesh(core_axis_name="c", subcore_axis_name="s")

    @functools.partial(
        pl.kernel, mesh=mesh,
        out_type=jax.ShapeDtypeStruct((T, H), jnp.float32),
        scratch_types=[
            pltpu.VMEM((K * toks_per_w,), jnp.int32),
            pltpu.VMEM((2, K * CT, Wp), jnp.int32),
            pltpu.VMEM((2, CT, H), jnp.float32),
            pltpu.SemaphoreType.DMA,
            pltpu.SemaphoreType.DMA,
        ],
    )
    def combine_k(rows_hbm, pos_hbm, out_hbm, idx_v, rows_v, acc_v, sem0, sem1):
        wid = lax.axis_index("s") * _NC + lax.axis_index("c")
        base = wid * toks_per_w
        pltpu.sync_copy(pos_hbm.at[pl.ds(K * base, K * toks_per_w)], idx_v)
        sems = (sem0, sem1)
        cps = [None, None]
        hi_mask = jnp.int32(-65536)

        def flush(j):
            b = j & 1
            cps[b].wait()

            def add_half(h0):
                def add_body(jj, c):
                    w0 = h0 + jj * 16
                    sl = pl.ds(w0, 16)
                    lo_sl = pl.ds(2 * h0 + jj * 16, 16)
                    hi_sl = pl.ds(2 * h0 + Qh + jj * 16, 16)
                    for t in range(CT):
                        v0 = rows_v[b, K * t, sl]
                        v1 = rows_v[b, K * t + 1, sl]
                        lo = (lax.bitcast_convert_type(
                                  jnp.left_shift(v0, 16), jnp.float32) +
                              lax.bitcast_convert_type(
                                  jnp.left_shift(v1, 16), jnp.float32))
                        hi = (lax.bitcast_convert_type(
                                  jnp.bitwise_and(v0, hi_mask), jnp.float32) +
                              lax.bitcast_convert_type(
                                  jnp.bitwise_and(v1, hi_mask), jnp.float32))
                        acc_v[b, t, lo_sl] = lo
                        acc_v[b, t, hi_sl] = hi
                    return c

                lax.fori_loop(0, Qh // 16, add_body, 0)

            for h0 in range(0, Wp, Qh):
                add_half(h0)
            pltpu.sync_copy(acc_v.at[b],
                            out_hbm.at[pl.ds(base + j * CT, CT)])

        for i in range(n_chunks):
            b = i & 1
            cps[b] = pltpu.async_copy(
                rows_hbm.at[idx_v.at[pl.ds(i * K * CT, K * CT)]],
                rows_v.at[b], sems[b])
            if i >= 1:
                flush(i - 1)
        flush(n_chunks - 1)

    return combine_k(out_rows, pos)


def _pack_rows(x):
    W = x.shape[1]
    b = x.astype(jnp.bfloat16)
    lo = lax.bitcast_convert_type(b[:, :W // 2], jnp.uint16).astype(jnp.uint32)
    hi = lax.bitcast_convert_type(b[:, W // 2:], jnp.uint16).astype(jnp.uint32)
    return lax.bitcast_convert_type(lo | (hi << 16), jnp.int32)


def _mlp_gate_up(xs32, Wg, Wu, gid, Pmax, H, I, NB):
    def body(gid_ref, xs_ref, wg_ref, wu_ref, hs_ref):
        x32 = xs_ref[...]
        x_lo = lax.bitcast_convert_type(jnp.left_shift(x32, 16), jnp.float32)
        hi_mask = jnp.int32(-65536)
        x_hi = lax.bitcast_convert_type(jnp.bitwise_and(x32, hi_mask),
                                        jnp.float32)
        dn = (((1,), (1,)), ((), ()))
        Hh = H // 2
        g = (lax.dot_general(x_lo, wg_ref[0][:, :Hh], dn,
                             preferred_element_type=jnp.float32) +
             lax.dot_general(x_hi, wg_ref[0][:, Hh:], dn,
                             preferred_element_type=jnp.float32))
        u = (lax.dot_general(x_lo, wu_ref[0][:, :Hh], dn,
                             preferred_element_type=jnp.float32) +
             lax.dot_general(x_hi, wu_ref[0][:, Hh:], dn,
                             preferred_element_type=jnp.float32))
        hs_ref[...] = ((g * jax.nn.sigmoid(g)) * u).astype(jnp.bfloat16)

    grid = (I // _BI, NB)
    spec = pltpu.PrefetchScalarGridSpec(
        num_scalar_prefetch=1,
        grid=grid,
        in_specs=[
            pl.BlockSpec((_BT, H // 2), lambda ib, nb, gid_ref: (nb, 0)),
            pl.BlockSpec((1, _BI, H), lambda ib, nb, gid_ref: (gid_ref[nb], ib, 0)),
            pl.BlockSpec((1, _BI, H), lambda ib, nb, gid_ref: (gid_ref[nb], ib, 0)),
        ],
        out_specs=pl.BlockSpec((_BT, _BI), lambda ib, nb, gid_ref: (nb, ib)),
    )
    return pl.pallas_call(
        body,
        grid_spec=spec,
        out_shape=jax.ShapeDtypeStruct((Pmax, I), jnp.bfloat16),
        compiler_params=pltpu.CompilerParams(
            dimension_semantics=("arbitrary", "arbitrary")),
    )(gid, xs32, Wg, Wu)


def _mlp_down(hs, Wd, w_row3, gid, Pmax, H, I, NB):
    def body(gid_ref, hs_ref, wd_ref, ws_ref, out_ref):
        dn = (((1,), (1,)), ((), ()))
        o = lax.dot_general(hs_ref[...], wd_ref[0], dn,
                            preferred_element_type=jnp.float32)
        o = o * ws_ref[0, 0][:, None]
        Bh = _BH // 2
        ob = o.astype(jnp.bfloat16)
        lo = lax.bitcast_convert_type(ob[:, :Bh], jnp.uint16).astype(jnp.uint32)
        hi = lax.bitcast_convert_type(ob[:, Bh:], jnp.uint16).astype(jnp.uint32)
        out_ref[...] = lax.bitcast_convert_type(lo | (hi << 16), jnp.int32)

    grid = (H // _BH, NB)
    spec = pltpu.PrefetchScalarGridSpec(
        num_scalar_prefetch=1,
        grid=grid,
        in_specs=[
            pl.BlockSpec((_BT, I), lambda hb, nb, gid_ref: (nb, 0)),
            pl.BlockSpec((1, _BH, I), lambda hb, nb, gid_ref: (gid_ref[nb], hb, 0)),
            pl.BlockSpec((1, 1, _BT), lambda hb, nb, gid_ref: (nb, 0, 0)),
        ],
        out_specs=pl.BlockSpec((_BT, _BH // 2), lambda hb, nb, gid_ref: (nb, hb)),
    )
    return pl.pallas_call(
        body,
        grid_spec=spec,
        out_shape=jax.ShapeDtypeStruct((Pmax, H // 2), jnp.int32),
        compiler_params=pltpu.CompilerParams(
            dimension_semantics=("arbitrary", "arbitrary")),
    )(gid, hs, Wd, w_row3)


def kernel(hidden_states, top_k_index, top_k_weights, Wg, Wu, Wd):
    T, H = hidden_states.shape
    K = top_k_index.shape[1]
    E, I, _ = Wg.shape
    Pmax = T * K + E * _BT
    NB = Pmax // _BT

    pos, src_tok, w_row, gid = _routing_metadata(
        top_k_index, top_k_weights, T, K, E, Pmax, NB)

    xs32 = _sc_gather(_pack_rows(hidden_states), src_tok, Pmax, H // 2,
                      jnp.int32)
    hs = _mlp_gate_up(xs32, Wg, Wu, gid, Pmax, H, I, NB)
    out_rows = _mlp_down(hs, Wd, w_row.reshape(NB, 1, _BT), gid, Pmax, H, I, NB)
    return _sc_combine(out_rows, pos, T, K, H)


# speedup vs baseline: 1.0084x; 1.0084x over previous
import functools

import jax
import jax.numpy as jnp
from jax import lax
from jax.experimental import pallas as pl
from jax.experimental.pallas import tpu as pltpu
from jax.experimental.pallas import tpu_sc as plsc

_NC = 2
_NS = 16
_NW = _NC * _NS

_BT = 256
_BI = 1024
_BH = 1024


def _routing_metadata(top_k_index, top_k_weights, T, K, E, Pmax, NB):
    A = T * K
    e_flat = top_k_index.reshape(A).astype(jnp.int32)
    w_flat = top_k_weights.reshape(A).astype(jnp.float32)
    onehot = (e_flat[:, None] == jnp.arange(E, dtype=jnp.int32)[None, :]).astype(jnp.int32)
    rank = jnp.take_along_axis(jnp.cumsum(onehot, axis=0) - onehot,
                               e_flat[:, None], axis=1)[:, 0]
    counts = jnp.sum(onehot, axis=0)
    blocks_per_e = (counts + _BT - 1) // _BT
    block_end = jnp.cumsum(blocks_per_e)
    group_start = (block_end - blocks_per_e) * _BT
    pos = group_start[e_flat] + rank
    tok = jnp.arange(A, dtype=jnp.int32) // K
    src_tok = jnp.zeros((Pmax,), jnp.int32).at[pos].set(tok)
    w_row = jnp.zeros((Pmax,), jnp.float32).at[pos].set(w_flat)
    gid = jnp.searchsorted(block_end, jnp.arange(NB), side="right")
    gid = jnp.minimum(gid, E - 1).astype(jnp.int32)
    return pos.astype(jnp.int32), src_tok, w_row, gid


def _sc_gather(table, src_tok, Pmax, W, dtype):
    rows_per_w = Pmax // _NW
    CH = 16
    n_chunks = rows_per_w // CH
    mesh = plsc.VectorSubcoreMesh(core_axis_name="c", subcore_axis_name="s")

    @functools.partial(
        pl.kernel, mesh=mesh,
        out_type=jax.ShapeDtypeStruct((Pmax, W), dtype),
        scratch_types=[
            pltpu.VMEM((rows_per_w,), jnp.int32),
            pltpu.VMEM((2, CH, W), dtype),
            pltpu.SemaphoreType.DMA,
            pltpu.SemaphoreType.DMA,
        ],
    )
    def gather_k(hs_hbm, idx_hbm, out_hbm, idx_v, rows_v, sem0, sem1):
        wid = lax.axis_index("s") * _NC + lax.axis_index("c")
        base = wid * rows_per_w
        pltpu.sync_copy(idx_hbm.at[pl.ds(base, rows_per_w)], idx_v)
        sems = (sem0, sem1)
        cps = [None, None]

        def flush(j):
            b = j & 1
            cps[b].wait()
            pltpu.sync_copy(rows_v.at[b], out_hbm.at[pl.ds(base + j * CH, CH)])

        for i in range(n_chunks):
            b = i & 1
            cps[b] = pltpu.async_copy(
                hs_hbm.at[idx_v.at[pl.ds(i * CH, CH)]], rows_v.at[b], sems[b])
            if i >= 1:
                flush(i - 1)
        flush(n_chunks - 1)

    return gather_k(table, src_tok)


def _sc_combine(out_rows, pos, T, K, H):
    toks_per_w = T // _NW
    CT = 8
    n_chunks = toks_per_w // CT
    Wp = H // 2
    Qh = _BH // 2
    mesh = plsc.VectorSubcoreMesh(core_axis_name="c", subcore_axis_name="s")

    @functools.partial(
        pl.kernel, mesh=mesh,
        out_type=jax.ShapeDtypeStruct((T, H), jnp.float32),
        scratch_types=[
            pltpu.VMEM((K * toks_per_w,), jnp.int32),
            pltpu.VMEM((2, K * CT, Wp), jnp.int32),
            pltpu.VMEM((2, CT, H), jnp.float32),
            pltpu.SemaphoreType.DMA,
            pltpu.SemaphoreType.DMA,
        ],
    )
    def combine_k(rows_hbm, pos_hbm, out_hbm, idx_v, rows_v, acc_v, sem0, sem1):
        wid = lax.axis_index("s") * _NC + lax.axis_index("c")
        base = wid * toks_per_w
        pltpu.sync_copy(pos_hbm.at[pl.ds(K * base, K * toks_per_w)], idx_v)
        sems = (sem0, sem1)
        cps = [None, None]
        hi_mask = jnp.int32(-65536)

        def flush(j):
            b = j & 1
            cps[b].wait()

            def add_half(h0):
                def add_body(jj, c):
                    w0 = h0 + jj * 16
                    sl = pl.ds(w0, 16)
                    lo_sl = pl.ds(2 * h0 + jj * 16, 16)
                    hi_sl = pl.ds(2 * h0 + Qh + jj * 16, 16)
                    for t in range(CT):
                        v0 = rows_v[b, K * t, sl]
                        v1 = rows_v[b, K * t + 1, sl]
                        lo = (lax.bitcast_convert_type(
                                  jnp.left_shift(v0, 16), jnp.float32) +
                              lax.bitcast_convert_type(
                                  jnp.left_shift(v1, 16), jnp.float32))
                        hi = (lax.bitcast_convert_type(
                                  jnp.bitwise_and(v0, hi_mask), jnp.float32) +
                              lax.bitcast_convert_type(
                                  jnp.bitwise_and(v1, hi_mask), jnp.float32))
                        acc_v[b, t, lo_sl] = lo
                        acc_v[b, t, hi_sl] = hi
                    return c

                lax.fori_loop(0, Qh // 16, add_body, 0)

            for h0 in range(0, Wp, Qh):
                add_half(h0)
            pltpu.sync_copy(acc_v.at[b],
                            out_hbm.at[pl.ds(base + j * CT, CT)])

        for i in range(n_chunks):
            b = i & 1
            cps[b] = pltpu.async_copy(
                rows_hbm.at[idx_v.at[pl.ds(i * K * CT, K * CT)]],
                rows_v.at[b], sems[b])
            if i >= 1:
                flush(i - 1)
        flush(n_chunks - 1)

    return combine_k(out_rows, pos)


def _pack_rows(x):
    W = x.shape[1]
    b = x.astype(jnp.bfloat16)
    lo = lax.bitcast_convert_type(b[:, :W // 2], jnp.uint16).astype(jnp.uint32)
    hi = lax.bitcast_convert_type(b[:, W // 2:], jnp.uint16).astype(jnp.uint32)
    return lax.bitcast_convert_type(lo | (hi << 16), jnp.int32)


def _mlp_gate_up(xs32, Wg, Wu, gid, Pmax, H, I, NB):
    def body(gid_ref, xs_ref, wg_ref, wu_ref, hs_ref):
        x32 = xs_ref[...]
        x_lo = lax.bitcast_convert_type(jnp.left_shift(x32, 16), jnp.float32)
        hi_mask = jnp.int32(-65536)
        x_hi = lax.bitcast_convert_type(jnp.bitwise_and(x32, hi_mask),
                                        jnp.float32)
        dn = (((1,), (1,)), ((), ()))
        Hh = H // 2
        g = (lax.dot_general(x_lo, wg_ref[0][:, :Hh], dn,
                             preferred_element_type=jnp.float32) +
             lax.dot_general(x_hi, wg_ref[0][:, Hh:], dn,
                             preferred_element_type=jnp.float32))
        u = (lax.dot_general(x_lo, wu_ref[0][:, :Hh], dn,
                             preferred_element_type=jnp.float32) +
             lax.dot_general(x_hi, wu_ref[0][:, Hh:], dn,
                             preferred_element_type=jnp.float32))
        hs_ref[...] = ((g * jax.nn.sigmoid(g)) * u).astype(jnp.bfloat16)

    grid = (I // _BI, NB)
    spec = pltpu.PrefetchScalarGridSpec(
        num_scalar_prefetch=1,
        grid=grid,
        in_specs=[
            pl.BlockSpec((_BT, H // 2), lambda ib, nb, gid_ref: (nb, 0)),
            pl.BlockSpec((1, _BI, H), lambda ib, nb, gid_ref: (gid_ref[nb], ib, 0)),
            pl.BlockSpec((1, _BI, H), lambda ib, nb, gid_ref: (gid_ref[nb], ib, 0)),
        ],
        out_specs=pl.BlockSpec((_BT, _BI), lambda ib, nb, gid_ref: (nb, ib)),
    )
    return pl.pallas_call(
        body,
        grid_spec=spec,
        out_shape=jax.ShapeDtypeStruct((Pmax, I), jnp.bfloat16),
        compiler_params=pltpu.CompilerParams(
            dimension_semantics=("arbitrary", "arbitrary")),
    )(gid, xs32, Wg, Wu)


def _mlp_down(hs, Wd, w_row3, gid, Pmax, H, I, NB):
    def body(gid_ref, hs_ref, wd_ref, ws_ref, out_ref):
        dn = (((1,), (1,)), ((), ()))
        o = lax.dot_general(hs_ref[...], wd_ref[0], dn,
                            preferred_element_type=jnp.float32)
        o = o * ws_ref[0, 0][:, None]
        Bh = _BH // 2
        ob = o.astype(jnp.bfloat16)
        lo = lax.bitcast_convert_type(ob[:, :Bh], jnp.uint16).astype(jnp.uint32)
        hi = lax.bitcast_convert_type(ob[:, Bh:], jnp.uint16).astype(jnp.uint32)
        out_ref[...] = lax.bitcast_convert_type(lo | (hi << 16), jnp.int32)

    grid = (H // _BH, NB)
    spec = pltpu.PrefetchScalarGridSpec(
        num_scalar_prefetch=1,
        grid=grid,
        in_specs=[
            pl.BlockSpec((_BT, I), lambda hb, nb, gid_ref: (nb, 0)),
            pl.BlockSpec((1, _BH, I), lambda hb, nb, gid_ref: (gid_ref[nb], hb, 0)),
            pl.BlockSpec((1, 1, _BT), lambda hb, nb, gid_ref: (nb, 0, 0)),
        ],
        out_specs=pl.BlockSpec((_BT, _BH // 2), lambda hb, nb, gid_ref: (nb, hb)),
    )
    return pl.pallas_call(
        body,
        grid_spec=spec,
        out_shape=jax.ShapeDtypeStruct((Pmax, H // 2), jnp.int32),
        compiler_params=pltpu.CompilerParams(
            dimension_semantics=("arbitrary", "arbitrary")),
    )(gid, hs, Wd, w_row3)


def kernel(hidden_states, top_k_index, top_k_weights, Wg, Wu, Wd):
    T, H = hidden_states.shape
    K = top_k_index.shape[1]
    E, I, _ = Wg.shape
    Pmax = T * K + E * _BT
    NB = Pmax // _BT

    pos, src_tok, w_row, gid = _routing_metadata(
        top_k_index, top_k_weights, T, K, E, Pmax, NB)

    xs32 = _sc_gather(_pack_rows(hidden_states), src_tok, Pmax, H // 2,
                      jnp.int32)
    hs = _mlp_gate_up(xs32, Wg, Wu, gid, Pmax, H, I, NB)
    out_rows = _mlp_down(hs, Wd, w_row.reshape(NB, 1, _BT), gid, Pmax, H, I, NB)
    return _sc_combine(out_rows, pos, T, K, H)
